# Initial kernel scaffold; baseline (speedup 1.0000x reference)
import jax
import jax.numpy as jnp
from jax.experimental import pallas as pl


def kernel(x_lc, edge_index, batch_lc, enc_W1, enc_b1, enc_W2, enc_b2, enc_W3, enc_b3, conv1_W, conv1_as, conv1_ad, conv1_b, norm1_g, norm1_beta, conv2_W, conv2_as, conv2_ad, conv2_b, norm2_g, norm2_beta, conv3_W, conv3_as, conv3_ad, conv3_b, norm3_g, norm3_beta, out_W1, out_b1, out_W2, out_b2, out_W3, out_b3):
    raise NotImplementedError("write your pallas kernel here")


# speedup vs baseline: 1.5920x; 1.5920x over previous
import functools

import jax
import jax.numpy as jnp
from jax import lax
from jax.experimental import pallas as pl
from jax.experimental.pallas import tpu as pltpu
from jax.experimental.pallas import tpu_sc as plsc

NC = 2
NS = 16
NW = NC * NS
L = 16

NP = 100352
NR = 8
R = NP // NR
TS = NP // NS
CH = 1024
NB = CH // 128
HID = 64
BN = 512
GN = NP // BN


def _elu(x):
    return jnp.where(x > 0, x, jnp.exp(x) - 1.0)


def _lrelu(x):
    return jnp.where(x >= 0, x, 0.2 * x)


def _full_spec(shape):
    nd = len(shape)
    return pl.BlockSpec(shape, lambda i, _nd=nd: (0,) * _nd)


def _node_spec(cols):
    return pl.BlockSpec((BN, cols), lambda i: (i, 0))


def _tc_encode(xp, w1, b1, w2, b2, w3, b3, cw, cas, cad):
    def body(x_ref, w1r, b1r, w2r, b2r, w3r, b3r, cwr, casr, cadr,
             r_ref, z_ref, asn_ref, adn_ref, sex_ref):
        x = x_ref[...]
        h = _elu(jnp.dot(x, w1r[...], preferred_element_type=jnp.float32) + b1r[...])
        h = _elu(jnp.dot(h, w2r[...], preferred_element_type=jnp.float32) + b2r[...])
        h = jnp.dot(h, w3r[...], preferred_element_type=jnp.float32) + b3r[...]
        z = jnp.dot(h, cwr[...], preferred_element_type=jnp.float32)
        a_s = jnp.dot(z, casr[...], preferred_element_type=jnp.float32)
        a_d = jnp.dot(z, cadr[...], preferred_element_type=jnp.float32)
        r_ref[...] = h
        z_ref[...] = z
        asn_ref[...] = a_s
        adn_ref[...] = a_d
        sex_ref[...] = jnp.exp(_lrelu(a_s + a_d))

    return pl.pallas_call(
        body,
        grid=(GN,),
        in_specs=[_node_spec(16)] + [_full_spec(a.shape)
                                     for a in (w1, b1, w2, b2, w3, b3, cw, cas, cad)],
        out_specs=[_node_spec(HID), _node_spec(HID),
                   _node_spec(1), _node_spec(1), _node_spec(1)],
        out_shape=[jax.ShapeDtypeStruct((NP, HID), jnp.float32),
                   jax.ShapeDtypeStruct((NP, HID), jnp.float32),
                   jax.ShapeDtypeStruct((NP, 1), jnp.float32),
                   jax.ShapeDtypeStruct((NP, 1), jnp.float32),
                   jax.ShapeDtypeStruct((NP, 1), jnp.float32)],
    )(xp, w1, b1, w2, b2, w3, b3, cw, cas, cad)


def _tc_mid(num, den0, den1, sex, z, r, cb, g, beta, nw, nas, nad):
    def body(num_ref, d0_ref, d1_ref, sex_ref, z_ref, r_ref, cb_ref, g_ref,
             b_ref, nw_ref, nas_ref, nad_ref,
             r2_ref, z2_ref, asn_ref, adn_ref, sex2_ref):
        sx = sex_ref[...]
        den = d0_ref[...] + d1_ref[...] + sx + 1e-16
        f = (num_ref[...] + sx * z_ref[...]) / den + cb_ref[...]
        y = f + r_ref[...]
        mu = jnp.mean(y, axis=1, keepdims=True)
        var = jnp.mean((y - mu) ** 2, axis=1, keepdims=True)
        xn = (y - mu) / jnp.sqrt(var + 1e-5) * g_ref[...] + b_ref[...]
        z2 = jnp.dot(xn, nw_ref[...], preferred_element_type=jnp.float32)
        a_s = jnp.dot(z2, nas_ref[...], preferred_element_type=jnp.float32)
        a_d = jnp.dot(z2, nad_ref[...], preferred_element_type=jnp.float32)
        r2_ref[...] = xn
        z2_ref[...] = z2
        asn_ref[...] = a_s
        adn_ref[...] = a_d
        sex2_ref[...] = jnp.exp(_lrelu(a_s + a_d))

    return pl.pallas_call(
        body,
        grid=(GN,),
        in_specs=[_node_spec(HID), _node_spec(1), _node_spec(1), _node_spec(1),
                  _node_spec(HID), _node_spec(HID)]
                 + [_full_spec(a.shape) for a in (cb, g, beta, nw, nas, nad)],
        out_specs=[_node_spec(HID), _node_spec(HID),
                   _node_spec(1), _node_spec(1), _node_spec(1)],
        out_shape=[jax.ShapeDtypeStruct((NP, HID), jnp.float32),
                   jax.ShapeDtypeStruct((NP, HID), jnp.float32),
                   jax.ShapeDtypeStruct((NP, 1), jnp.float32),
                   jax.ShapeDtypeStruct((NP, 1), jnp.float32),
                   jax.ShapeDtypeStruct((NP, 1), jnp.float32)],
    )(num, den0, den1, sex, z, r, cb, g, beta, nw, nas, nad)


def _tc_post(num, den0, den1, sex, z, r, cb, g, beta, ow1, ob1, ow2, ob2, ow3, ob3):
    def body(num_ref, d0_ref, d1_ref, sex_ref, z_ref, r_ref, cb_ref, g_ref,
             b_ref, w1r, b1r, w2r, b2r, w3r, b3r, o_ref):
        sx = sex_ref[...]
        den = d0_ref[...] + d1_ref[...] + sx + 1e-16
        f = (num_ref[...] + sx * z_ref[...]) / den + cb_ref[...]
        y = f + r_ref[...]
        mu = jnp.mean(y, axis=1, keepdims=True)
        var = jnp.mean((y - mu) ** 2, axis=1, keepdims=True)
        xn = (y - mu) / jnp.sqrt(var + 1e-5) * g_ref[...] + b_ref[...]
        o = _elu(jnp.dot(xn, w1r[...], preferred_element_type=jnp.float32) + b1r[...])
        o = _elu(jnp.dot(o, w2r[...], preferred_element_type=jnp.float32) + b2r[...])
        o_ref[...] = jnp.dot(o, w3r[...], preferred_element_type=jnp.float32) + b3r[...]

    return pl.pallas_call(
        body,
        grid=(GN,),
        in_specs=[_node_spec(HID), _node_spec(1), _node_spec(1), _node_spec(1),
                  _node_spec(HID), _node_spec(HID)]
                 + [_full_spec(a.shape)
                    for a in (cb, g, beta, ow1, ob1, ow2, ob2, ow3, ob3)],
        out_specs=[_node_spec(8)],
        out_shape=[jax.ShapeDtypeStruct((NP, 8), jnp.float32)],
    )(num, den0, den1, sex, z, r, cb, g, beta, ow1, ob1, ow2, ob2, ow3, ob3)[0]


def _sc_edge_ex(src, dst, asn, adn, kw):
    ep = src.shape[0]
    mesh = plsc.VectorSubcoreMesh(core_axis_name="c", subcore_axis_name="s",
                                  num_cores=NC, num_subcores=NS)

    @functools.partial(
        pl.kernel,
        out_type=[jax.ShapeDtypeStruct((ep,), jnp.float32),
                  jax.ShapeDtypeStruct((NC, NP), jnp.float32)],
        mesh=mesh,
        compiler_params=pltpu.CompilerParams(needs_layout_passes=False),
        scratch_types=[
            pltpu.VMEM((NP,), jnp.float32),
            pltpu.VMEM((CH,), jnp.int32),
            pltpu.VMEM((CH,), jnp.int32),
            pltpu.VMEM((NB, 128), jnp.int32),
            pltpu.VMEM((CH,), jnp.float32),
            pltpu.VMEM((CH,), jnp.float32),
            pltpu.VMEM((TS,), jnp.float32),
            pltpu.VMEM_SHARED((NP,), jnp.float32),
            pltpu.SemaphoreType.DMA,
        ],
    )
    def k(src_hbm, dst_hbm, asn_hbm, adn_hbm, ex_hbm, den_hbm,
          asn_v, srcl, dstl, dst2d, adv, exl, zb, den_sh, sem):
        c = lax.axis_index("c")
        s = lax.axis_index("s")
        w = s * NC + c

        def zi(i, _):
            zb[pl.ds(i * L, L)] = jnp.zeros((L,), jnp.float32)
            return 0
        lax.fori_loop(0, TS // L, zi, 0)
        pltpu.sync_copy(zb, den_sh.at[pl.ds(s * TS, TS)])
        pltpu.sync_copy(asn_hbm, asn_v)
        plsc.subcore_barrier()

        def chunk(i, _):
            c0 = (w * kw + i) * CH
            pltpu.sync_copy(src_hbm.at[pl.ds(c0, CH)], srcl)
            pltpu.sync_copy(dst_hbm.at[pl.ds(c0, CH)], dstl)

            def stage(b, _):
                pltpu.sync_copy(dst_hbm.at[pl.ds(c0 + b * 128, 128)], dst2d.at[b])
                pltpu.async_copy(adn_hbm.at[dstl.at[pl.ds(b * 128, 128)]],
                                 adv.at[pl.ds(b * 128, 128)], sem).wait()
                return 0
            lax.fori_loop(0, NB, stage, 0)

            def grp(gi, _):
                s16 = srcl[pl.ds(gi * L, L)]
                a_s = plsc.load_gather(asn_v, [s16])
                e16 = a_s + adv[pl.ds(gi * L, L)]
                e16 = jnp.where(e16 >= 0, e16, 0.2 * e16)
                exl[pl.ds(gi * L, L)] = jnp.exp(e16)
                return 0
            lax.fori_loop(0, CH // L, grp, 0)

            pltpu.sync_copy(exl, ex_hbm.at[pl.ds(c0, CH)])

            def dadd(b, _):
                pltpu.sync_copy(exl.at[pl.ds(b * 128, 128)],
                                den_sh.at[dst2d.at[b]], add=True)
                return 0
            lax.fori_loop(0, NB, dadd, 0)
            return 0
        lax.fori_loop(0, kw, chunk, 0)

        plsc.subcore_barrier()
        pltpu.sync_copy(den_sh.at[pl.ds(s * TS, TS)],
                        den_hbm.at[c, pl.ds(s * TS, TS)])

    return k(src, dst, asn, adn)


def _sc_aggregate(src, dst, ex, z, zrows):
    ep = src.shape[0]
    ncht = ep // CH // NS
    mesh = plsc.VectorSubcoreMesh(core_axis_name="c", subcore_axis_name="s",
                                  num_cores=NC, num_subcores=NS)

    @functools.partial(
        pl.kernel,
        out_type=jax.ShapeDtypeStruct((NP, HID), jnp.float32),
        mesh=mesh,
        compiler_params=pltpu.CompilerParams(needs_layout_passes=False,
                                             use_tc_tiling_on_sc=False),
        scratch_types=[
            pltpu.VMEM((CH,), jnp.int32),
            pltpu.VMEM((CH,), jnp.int32),
            pltpu.VMEM((CH,), jnp.float32),
            pltpu.VMEM((CH,), jnp.float32),
            pltpu.VMEM((NB, 128), jnp.int32),
            pltpu.VMEM((CH, HID), jnp.float32),
            pltpu.VMEM((16, HID), jnp.float32),
            pltpu.VMEM_SHARED((R, HID), jnp.float32),
            pltpu.SemaphoreType.DMA,
        ],
    )
    def k(src_hbm, dst_hbm, ex_hbm, z_hbm, zr_hbm, num_hbm,
          srcl, dstl, exl, wbuf, off2d, rows, zb, acc, sem):
        c = lax.axis_index("c")
        s = lax.axis_index("s")
        pltpu.sync_copy(zr_hbm, zb)

        def rngloop(rng, _):
            g = c * (NR // NC) + rng
            base = g * R

            def zi(i, _):
                pltpu.sync_copy(zb, acc.at[pl.ds(s * (R // NS) + i * 16, 16)])
                return 0
            lax.fori_loop(0, R // NS // 16, zi, 0)
            plsc.subcore_barrier()

            def chunk(i, _):
                c0 = (s * ncht + i) * CH
                pltpu.sync_copy(src_hbm.at[pl.ds(c0, CH)], srcl)
                pltpu.sync_copy(dst_hbm.at[pl.ds(c0, CH)], dstl)
                pltpu.sync_copy(ex_hbm.at[pl.ds(c0, CH)], exl)

                def batch(b, _):
                    gd = pltpu.async_copy(
                        z_hbm.at[srcl.at[pl.ds(b * 128, 128)]],
                        rows.at[pl.ds(b * 128, 128)], sem)

                    def grp(j, _):
                        p0 = b * 128 + j * L
                        d16 = dstl[pl.ds(p0, L)]
                        e16 = exl[pl.ds(p0, L)]
                        inr = (d16 >= base) & (d16 < base + R)
                        off = jnp.where(inr, d16 - base,
                                        jnp.bitwise_and(d16, 8191))
                        wbuf[pl.ds(p0, L)] = jnp.where(inr, e16, 0.0)
                        off2d.at[b][pl.ds(j * L, L)] = off
                        return 0
                    lax.fori_loop(0, 128 // L, grp, 0)
                    gd.wait()

                    def scale(j, _):
                        p0 = b * 128 + j * L
                        w16 = wbuf[pl.ds(p0, L)]
                        e16 = p0 + lax.iota(jnp.int32, L)
                        for cc in range(HID):
                            ci = jnp.full((L,), cc, jnp.int32)
                            v = plsc.load_gather(rows, [e16, ci])
                            plsc.store_scatter(rows, [e16, ci], v * w16)
                        return 0
                    lax.fori_loop(0, 128 // L, scale, 0)

                    pltpu.sync_copy(rows.at[pl.ds(b * 128, 128)],
                                    acc.at[off2d.at[b]], add=True)
                    return 0
                lax.fori_loop(0, NB, batch, 0)
                return 0
            lax.fori_loop(0, ncht, chunk, 0)

            plsc.subcore_barrier()
            pltpu.sync_copy(acc.at[pl.ds(s * (R // NS), R // NS)],
                            num_hbm.at[pl.ds(base + s * (R // NS), R // NS)])
            plsc.subcore_barrier()
            return 0
        lax.fori_loop(0, NR // NC, rngloop, 0)

    return k(src, dst, ex, z, zrows)


def kernel(x_lc, edge_index, batch_lc, enc_W1, enc_b1, enc_W2, enc_b2, enc_W3,
           enc_b3, conv1_W, conv1_as, conv1_ad, conv1_b, norm1_g, norm1_beta,
           conv2_W, conv2_as, conv2_ad, conv2_b, norm2_g, norm2_beta, conv3_W,
           conv3_as, conv3_ad, conv3_b, norm3_g, norm3_beta, out_W1, out_b1,
           out_W2, out_b2, out_W3, out_b3):
    n = x_lc.shape[0]
    e = edge_index.shape[1]
    kw = -(-e // (NW * CH))
    ep = NW * kw * CH

    src = edge_index[0]
    dst = edge_index[1]
    pid = jnp.arange(ep - e, dtype=jnp.int32)
    src_p = jnp.concatenate([src, pid % 1024])
    dst_p = jnp.concatenate([dst, n + (pid % 64)])

    xp = jnp.pad(x_lc, ((0, NP - n), (0, 1)))
    w1 = jnp.pad(enc_W1, ((0, 1), (0, 0)))
    row = lambda v: v.reshape(1, -1)
    col = lambda v: v.reshape(-1, 1)
    zrows = jnp.zeros((16, HID), jnp.float32)

    r, z, asn, adn, sex = _tc_encode(
        xp, w1, row(enc_b1), enc_W2, row(enc_b2), enc_W3, row(enc_b3),
        conv1_W, col(conv1_as), col(conv1_ad))

    layers = [
        (conv1_b, norm1_g, norm1_beta, conv2_W, conv2_as, conv2_ad),
        (conv2_b, norm2_g, norm2_beta, conv3_W, conv3_as, conv3_ad),
    ]
    for cb, g, beta, nw_, nas, nad in layers:
        ex, den = _sc_edge_ex(src_p, dst_p, asn.reshape(NP), adn.reshape(NP), kw)
        num = _sc_aggregate(src_p, dst_p, ex, z, zrows)
        r, z, asn, adn, sex = _tc_mid(
            num, col(den[0]), col(den[1]), sex, z, r,
            row(cb), row(g), row(beta), nw_, col(nas), col(nad))

    ex, den = _sc_edge_ex(src_p, dst_p, asn.reshape(NP), adn.reshape(NP), kw)
    num = _sc_aggregate(src_p, dst_p, ex, z, zrows)
    o = _tc_post(num, col(den[0]), col(den[1]), sex, z, r,
                 row(conv3_b), row(norm3_g), row(norm3_beta),
                 out_W1, row(out_b1), out_W2, row(out_b2), out_W3, row(out_b3))

    return (o[:n], batch_lc)


# speedup vs baseline: 8.7099x; 5.4710x over previous
import functools

import jax
import jax.numpy as jnp
from jax import lax
from jax.experimental import pallas as pl
from jax.experimental.pallas import tpu as pltpu
from jax.experimental.pallas import tpu_sc as plsc

NC = 2
NS = 16
NW = NC * NS
L = 16

NP = 100352
NR = 4
R = NP // NR
CH2 = 512
PSZ = CH2 + 128 + L
TS = NP // NS
CH = 1024
NB = CH // 128
HID = 64
BN = 512
GN = NP // BN


def _elu(x):
    return jnp.where(x > 0, x, jnp.exp(x) - 1.0)


def _lrelu(x):
    return jnp.where(x >= 0, x, 0.2 * x)


def _full_spec(shape):
    nd = len(shape)
    return pl.BlockSpec(shape, lambda i, _nd=nd: (0,) * _nd)


def _node_spec(cols):
    return pl.BlockSpec((BN, cols), lambda i: (i, 0))


def _tc_encode(xp, w1, b1, w2, b2, w3, b3, cw, cas, cad):
    def body(x_ref, w1r, b1r, w2r, b2r, w3r, b3r, cwr, casr, cadr,
             r_ref, z_ref, asn_ref, adn_ref, sex_ref):
        x = x_ref[...]
        h = _elu(jnp.dot(x, w1r[...], preferred_element_type=jnp.float32) + b1r[...])
        h = _elu(jnp.dot(h, w2r[...], preferred_element_type=jnp.float32) + b2r[...])
        h = jnp.dot(h, w3r[...], preferred_element_type=jnp.float32) + b3r[...]
        z = jnp.dot(h, cwr[...], preferred_element_type=jnp.float32)
        a_s = jnp.dot(z, casr[...], preferred_element_type=jnp.float32)
        a_d = jnp.dot(z, cadr[...], preferred_element_type=jnp.float32)
        r_ref[...] = h
        z_ref[...] = z
        asn_ref[...] = a_s
        adn_ref[...] = a_d
        sex_ref[...] = jnp.exp(_lrelu(a_s + a_d))

    return pl.pallas_call(
        body,
        grid=(GN,),
        in_specs=[_node_spec(16)] + [_full_spec(a.shape)
                                     for a in (w1, b1, w2, b2, w3, b3, cw, cas, cad)],
        out_specs=[_node_spec(HID), _node_spec(HID),
                   _node_spec(1), _node_spec(1), _node_spec(1)],
        out_shape=[jax.ShapeDtypeStruct((NP, HID), jnp.float32),
                   jax.ShapeDtypeStruct((NP, HID), jnp.float32),
                   jax.ShapeDtypeStruct((NP, 1), jnp.float32),
                   jax.ShapeDtypeStruct((NP, 1), jnp.float32),
                   jax.ShapeDtypeStruct((NP, 1), jnp.float32)],
    )(xp, w1, b1, w2, b2, w3, b3, cw, cas, cad)


def _tc_mid(num, den0, den1, sex, z, r, cb, g, beta, nw, nas, nad):
    def body(num_ref, d0_ref, d1_ref, sex_ref, z_ref, r_ref, cb_ref, g_ref,
             b_ref, nw_ref, nas_ref, nad_ref,
             r2_ref, z2_ref, asn_ref, adn_ref, sex2_ref):
        sx = sex_ref[...]
        den = d0_ref[...] + d1_ref[...] + sx + 1e-16
        f = (num_ref[...] + sx * z_ref[...]) / den + cb_ref[...]
        y = f + r_ref[...]
        mu = jnp.mean(y, axis=1, keepdims=True)
        var = jnp.mean((y - mu) ** 2, axis=1, keepdims=True)
        xn = (y - mu) / jnp.sqrt(var + 1e-5) * g_ref[...] + b_ref[...]
        z2 = jnp.dot(xn, nw_ref[...], preferred_element_type=jnp.float32)
        a_s = jnp.dot(z2, nas_ref[...], preferred_element_type=jnp.float32)
        a_d = jnp.dot(z2, nad_ref[...], preferred_element_type=jnp.float32)
        r2_ref[...] = xn
        z2_ref[...] = z2
        asn_ref[...] = a_s
        adn_ref[...] = a_d
        sex2_ref[...] = jnp.exp(_lrelu(a_s + a_d))

    return pl.pallas_call(
        body,
        grid=(GN,),
        in_specs=[_node_spec(HID), _node_spec(1), _node_spec(1), _node_spec(1),
                  _node_spec(HID), _node_spec(HID)]
                 + [_full_spec(a.shape) for a in (cb, g, beta, nw, nas, nad)],
        out_specs=[_node_spec(HID), _node_spec(HID),
                   _node_spec(1), _node_spec(1), _node_spec(1)],
        out_shape=[jax.ShapeDtypeStruct((NP, HID), jnp.float32),
                   jax.ShapeDtypeStruct((NP, HID), jnp.float32),
                   jax.ShapeDtypeStruct((NP, 1), jnp.float32),
                   jax.ShapeDtypeStruct((NP, 1), jnp.float32),
                   jax.ShapeDtypeStruct((NP, 1), jnp.float32)],
    )(num, den0, den1, sex, z, r, cb, g, beta, nw, nas, nad)


def _tc_post(num, den0, den1, sex, z, r, cb, g, beta, ow1, ob1, ow2, ob2, ow3, ob3):
    def body(num_ref, d0_ref, d1_ref, sex_ref, z_ref, r_ref, cb_ref, g_ref,
             b_ref, w1r, b1r, w2r, b2r, w3r, b3r, o_ref):
        sx = sex_ref[...]
        den = d0_ref[...] + d1_ref[...] + sx + 1e-16
        f = (num_ref[...] + sx * z_ref[...]) / den + cb_ref[...]
        y = f + r_ref[...]
        mu = jnp.mean(y, axis=1, keepdims=True)
        var = jnp.mean((y - mu) ** 2, axis=1, keepdims=True)
        xn = (y - mu) / jnp.sqrt(var + 1e-5) * g_ref[...] + b_ref[...]
        o = _elu(jnp.dot(xn, w1r[...], preferred_element_type=jnp.float32) + b1r[...])
        o = _elu(jnp.dot(o, w2r[...], preferred_element_type=jnp.float32) + b2r[...])
        o_ref[...] = jnp.dot(o, w3r[...], preferred_element_type=jnp.float32) + b3r[...]

    return pl.pallas_call(
        body,
        grid=(GN,),
        in_specs=[_node_spec(HID), _node_spec(1), _node_spec(1), _node_spec(1),
                  _node_spec(HID), _node_spec(HID)]
                 + [_full_spec(a.shape)
                    for a in (cb, g, beta, ow1, ob1, ow2, ob2, ow3, ob3)],
        out_specs=[_node_spec(8)],
        out_shape=[jax.ShapeDtypeStruct((NP, 8), jnp.float32)],
    )(num, den0, den1, sex, z, r, cb, g, beta, ow1, ob1, ow2, ob2, ow3, ob3)[0]


def _sc_edge_ex(src, dst, asn, adn, kw):
    ep = src.shape[0]
    mesh = plsc.VectorSubcoreMesh(core_axis_name="c", subcore_axis_name="s",
                                  num_cores=NC, num_subcores=NS)

    @functools.partial(
        pl.kernel,
        out_type=[jax.ShapeDtypeStruct((ep,), jnp.float32),
                  jax.ShapeDtypeStruct((NC, NP), jnp.float32)],
        mesh=mesh,
        compiler_params=pltpu.CompilerParams(needs_layout_passes=False),
        scratch_types=[
            pltpu.VMEM((NP,), jnp.float32),
            pltpu.VMEM((CH,), jnp.int32),
            pltpu.VMEM((CH,), jnp.int32),
            pltpu.VMEM((NB, 128), jnp.int32),
            pltpu.VMEM((CH,), jnp.float32),
            pltpu.VMEM((CH,), jnp.float32),
            pltpu.VMEM((TS,), jnp.float32),
            pltpu.VMEM_SHARED((NP,), jnp.float32),
            pltpu.SemaphoreType.DMA,
        ],
    )
    def k(src_hbm, dst_hbm, asn_hbm, adn_hbm, ex_hbm, den_hbm,
          asn_v, srcl, dstl, dst2d, adv, exl, zb, den_sh, sem):
        c = lax.axis_index("c")
        s = lax.axis_index("s")
        w = s * NC + c

        def zi(i, _):
            zb[pl.ds(i * L, L)] = jnp.zeros((L,), jnp.float32)
            return 0
        lax.fori_loop(0, TS // L, zi, 0)
        pltpu.sync_copy(zb, den_sh.at[pl.ds(s * TS, TS)])
        pltpu.sync_copy(asn_hbm, asn_v)
        plsc.subcore_barrier()

        def chunk(i, _):
            c0 = (w * kw + i) * CH
            pltpu.sync_copy(src_hbm.at[pl.ds(c0, CH)], srcl)
            pltpu.sync_copy(dst_hbm.at[pl.ds(c0, CH)], dstl)

            def stage(b, _):
                pltpu.sync_copy(dst_hbm.at[pl.ds(c0 + b * 128, 128)], dst2d.at[b])
                pltpu.async_copy(adn_hbm.at[dstl.at[pl.ds(b * 128, 128)]],
                                 adv.at[pl.ds(b * 128, 128)], sem).wait()
                return 0
            lax.fori_loop(0, NB, stage, 0)

            def grp(gi, _):
                s16 = srcl[pl.ds(gi * L, L)]
                a_s = plsc.load_gather(asn_v, [s16])
                e16 = a_s + adv[pl.ds(gi * L, L)]
                e16 = jnp.where(e16 >= 0, e16, 0.2 * e16)
                exl[pl.ds(gi * L, L)] = jnp.exp(e16)
                return 0
            lax.fori_loop(0, CH // L, grp, 0)

            pltpu.sync_copy(exl, ex_hbm.at[pl.ds(c0, CH)])

            def dadd(b, _):
                pltpu.sync_copy(exl.at[pl.ds(b * 128, 128)],
                                den_sh.at[dst2d.at[b]], add=True)
                return 0
            lax.fori_loop(0, NB, dadd, 0)
            return 0
        lax.fori_loop(0, kw, chunk, 0)

        plsc.subcore_barrier()
        pltpu.sync_copy(den_sh.at[pl.ds(s * TS, TS)],
                        den_hbm.at[c, pl.ds(s * TS, TS)])

    return k(src, dst, asn, adn)


def _sc_aggregate(src, dst, ex, z, zrows):
    ep = src.shape[0]
    ncht = ep // CH2 // NS
    mesh = plsc.VectorSubcoreMesh(core_axis_name="c", subcore_axis_name="s",
                                  num_cores=NC, num_subcores=NS)

    @functools.partial(
        pl.kernel,
        out_type=jax.ShapeDtypeStruct((NP, HID), jnp.float32),
        mesh=mesh,
        compiler_params=pltpu.CompilerParams(needs_layout_passes=False,
                                             use_tc_tiling_on_sc=False),
        scratch_types=[
            pltpu.VMEM((CH2,), jnp.int32),
            pltpu.VMEM((CH2,), jnp.int32),
            pltpu.VMEM((CH2,), jnp.float32),
            pltpu.VMEM((PSZ,), jnp.int32),
            pltpu.VMEM((PSZ,), jnp.int32),
            pltpu.VMEM((PSZ,), jnp.float32),
            pltpu.VMEM((128,), jnp.int32),
            pltpu.VMEM((128,), jnp.int32),
            pltpu.VMEM((128,), jnp.float32),
            pltpu.VMEM((128, HID), jnp.float32),
            pltpu.VMEM((16, HID), jnp.float32),
            pltpu.VMEM_SHARED((R, HID), jnp.float32),
            pltpu.SemaphoreType.DMA,
        ],
    )
    def k(src_hbm, dst_hbm, ex_hbm, z_hbm, zr_hbm, num_hbm,
          srcl, dstl, exl, psrc, poff, pex, sidx, obat, wbat, rows, zb,
          acc, sem):
        c = lax.axis_index("c")
        s = lax.axis_index("s")
        pltpu.sync_copy(zr_hbm, zb)

        def fire(qbase, limit):
            def bg(k_, _):
                pos = qbase + k_ * L
                lid = pos + lax.iota(jnp.int32, L)
                valid = lid < limit
                spread = (k_ * L) + lax.iota(jnp.int32, L)
                sidx[pl.ds(k_ * L, L)] = jnp.where(valid, psrc[pl.ds(pos, L)],
                                                   spread)
                obat[pl.ds(k_ * L, L)] = jnp.where(valid, poff[pl.ds(pos, L)],
                                                   spread)
                wbat[pl.ds(k_ * L, L)] = jnp.where(valid, pex[pl.ds(pos, L)],
                                                   0.0)
                return 0
            lax.fori_loop(0, 128 // L, bg, 0)
            gd = pltpu.async_copy(z_hbm.at[sidx], rows, sem)
            gd.wait()

            def scale(j, _):
                w16 = wbat[pl.ds(j * L, L)]
                e16 = j * L + lax.iota(jnp.int32, L)
                for cc in range(HID):
                    ci = jnp.full((L,), cc, jnp.int32)
                    v = plsc.load_gather(rows, [e16, ci])
                    plsc.store_scatter(rows, [e16, ci], v * w16)
                return 0
            lax.fori_loop(0, 128 // L, scale, 0)
            pltpu.sync_copy(rows, acc.at[obat], add=True)

        def rngloop(rng, _):
            g = c * (NR // NC) + rng
            base = g * R

            def zi(i, _):
                pltpu.sync_copy(zb, acc.at[pl.ds(s * (R // NS) + i * 16, 16)])
                return 0
            lax.fori_loop(0, R // NS // 16, zi, 0)
            plsc.subcore_barrier()

            def chunk(i, pcount):
                c0 = (s * ncht + i) * CH2
                pltpu.sync_copy(src_hbm.at[pl.ds(c0, CH2)], srcl)
                pltpu.sync_copy(dst_hbm.at[pl.ds(c0, CH2)], dstl)
                pltpu.sync_copy(ex_hbm.at[pl.ds(c0, CH2)], exl)

                def cmp(j, p):
                    d16 = dstl[pl.ds(j * L, L)]
                    inr = (d16 >= base) & (d16 < base + R)
                    plsc.store_compressed(psrc.at[pl.ds(p, L)],
                                          srcl[pl.ds(j * L, L)], mask=inr)
                    plsc.store_compressed(poff.at[pl.ds(p, L)],
                                          d16 - base, mask=inr)
                    plsc.store_compressed(pex.at[pl.ds(p, L)],
                                          exl[pl.ds(j * L, L)], mask=inr)
                    return p + jnp.sum(inr.astype(jnp.int32))
                p1 = lax.fori_loop(0, CH2 // L, cmp, pcount)

                def drain_cond(q):
                    return p1 - q >= 128

                def drain(q):
                    fire(q, p1)
                    return q + 128
                q1 = lax.while_loop(drain_cond, drain, jnp.int32(0))

                def mv(k_, _):
                    psrc[pl.ds(k_ * L, L)] = psrc[pl.ds(q1 + k_ * L, L)]
                    poff[pl.ds(k_ * L, L)] = poff[pl.ds(q1 + k_ * L, L)]
                    pex[pl.ds(k_ * L, L)] = pex[pl.ds(q1 + k_ * L, L)]
                    return 0
                lax.fori_loop(0, 128 // L, mv, 0)
                return p1 - q1
            pfin = lax.fori_loop(0, ncht, chunk, jnp.int32(0))
            fire(jnp.int32(0), pfin)

            plsc.subcore_barrier()
            pltpu.sync_copy(acc.at[pl.ds(s * (R // NS), R // NS)],
                            num_hbm.at[pl.ds(base + s * (R // NS), R // NS)])
            plsc.subcore_barrier()
            return 0
        lax.fori_loop(0, NR // NC, rngloop, 0)

    return k(src, dst, ex, z, zrows)


def kernel(x_lc, edge_index, batch_lc, enc_W1, enc_b1, enc_W2, enc_b2, enc_W3,
           enc_b3, conv1_W, conv1_as, conv1_ad, conv1_b, norm1_g, norm1_beta,
           conv2_W, conv2_as, conv2_ad, conv2_b, norm2_g, norm2_beta, conv3_W,
           conv3_as, conv3_ad, conv3_b, norm3_g, norm3_beta, out_W1, out_b1,
           out_W2, out_b2, out_W3, out_b3):
    n = x_lc.shape[0]
    e = edge_index.shape[1]
    kw = -(-e // (NW * CH))
    ep = NW * kw * CH

    src = edge_index[0]
    dst = edge_index[1]
    pid = jnp.arange(ep - e, dtype=jnp.int32)
    src_p = jnp.concatenate([src, pid % 1024])
    dst_p = jnp.concatenate([dst, n + (pid % 64)])

    xp = jnp.pad(x_lc, ((0, NP - n), (0, 1)))
    w1 = jnp.pad(enc_W1, ((0, 1), (0, 0)))
    row = lambda v: v.reshape(1, -1)
    col = lambda v: v.reshape(-1, 1)
    zrows = jnp.zeros((16, HID), jnp.float32)

    r, z, asn, adn, sex = _tc_encode(
        xp, w1, row(enc_b1), enc_W2, row(enc_b2), enc_W3, row(enc_b3),
        conv1_W, col(conv1_as), col(conv1_ad))

    layers = [
        (conv1_b, norm1_g, norm1_beta, conv2_W, conv2_as, conv2_ad),
        (conv2_b, norm2_g, norm2_beta, conv3_W, conv3_as, conv3_ad),
    ]
    for cb, g, beta, nw_, nas, nad in layers:
        ex, den = _sc_edge_ex(src_p, dst_p, asn.reshape(NP), adn.reshape(NP), kw)
        num = _sc_aggregate(src_p, dst_p, ex, z, zrows)
        r, z, asn, adn, sex = _tc_mid(
            num, col(den[0]), col(den[1]), sex, z, r,
            row(cb), row(g), row(beta), nw_, col(nas), col(nad))

    ex, den = _sc_edge_ex(src_p, dst_p, asn.reshape(NP), adn.reshape(NP), kw)
    num = _sc_aggregate(src_p, dst_p, ex, z, zrows)
    o = _tc_post(num, col(den[0]), col(den[1]), sex, z, r,
                 row(conv3_b), row(norm3_g), row(norm3_beta),
                 out_W1, row(out_b1), out_W2, row(out_b2), out_W3, row(out_b3))

    return (o[:n], batch_lc)


# speedup vs baseline: 9.2288x; 1.0596x over previous
import functools

import jax
import jax.numpy as jnp
from jax import lax
from jax.experimental import pallas as pl
from jax.experimental.pallas import tpu as pltpu
from jax.experimental.pallas import tpu_sc as plsc

NC = 2
NS = 16
NW = NC * NS
L = 16

NP = 100352
NR = 4
R = NP // NR
CH2 = 1024
FB = 256
PSZ = CH2 + FB + L
TS = NP // NS
CH = 1024
NB = CH // 128
HID = 64
BN = 512
GN = NP // BN


def _elu(x):
    return jnp.where(x > 0, x, jnp.exp(x) - 1.0)


def _lrelu(x):
    return jnp.where(x >= 0, x, 0.2 * x)


def _full_spec(shape):
    nd = len(shape)
    return pl.BlockSpec(shape, lambda i, _nd=nd: (0,) * _nd)


def _node_spec(cols):
    return pl.BlockSpec((BN, cols), lambda i: (i, 0))


def _tc_encode(xp, w1, b1, w2, b2, w3, b3, cw, cas, cad):
    def body(x_ref, w1r, b1r, w2r, b2r, w3r, b3r, cwr, casr, cadr,
             r_ref, z_ref, asn_ref, adn_ref, sex_ref):
        x = x_ref[...]
        h = _elu(jnp.dot(x, w1r[...], preferred_element_type=jnp.float32) + b1r[...])
        h = _elu(jnp.dot(h, w2r[...], preferred_element_type=jnp.float32) + b2r[...])
        h = jnp.dot(h, w3r[...], preferred_element_type=jnp.float32) + b3r[...]
        z = jnp.dot(h, cwr[...], preferred_element_type=jnp.float32)
        a_s = jnp.dot(z, casr[...], preferred_element_type=jnp.float32)
        a_d = jnp.dot(z, cadr[...], preferred_element_type=jnp.float32)
        r_ref[...] = h
        z_ref[...] = z
        asn_ref[...] = a_s
        adn_ref[...] = a_d
        sex_ref[...] = jnp.exp(_lrelu(a_s + a_d))

    return pl.pallas_call(
        body,
        grid=(GN,),
        in_specs=[_node_spec(16)] + [_full_spec(a.shape)
                                     for a in (w1, b1, w2, b2, w3, b3, cw, cas, cad)],
        out_specs=[_node_spec(HID), _node_spec(HID),
                   _node_spec(1), _node_spec(1), _node_spec(1)],
        out_shape=[jax.ShapeDtypeStruct((NP, HID), jnp.float32),
                   jax.ShapeDtypeStruct((NP, HID), jnp.float32),
                   jax.ShapeDtypeStruct((NP, 1), jnp.float32),
                   jax.ShapeDtypeStruct((NP, 1), jnp.float32),
                   jax.ShapeDtypeStruct((NP, 1), jnp.float32)],
    )(xp, w1, b1, w2, b2, w3, b3, cw, cas, cad)


def _tc_mid(num, den0, den1, sex, z, r, cb, g, beta, nw, nas, nad):
    def body(num_ref, d0_ref, d1_ref, sex_ref, z_ref, r_ref, cb_ref, g_ref,
             b_ref, nw_ref, nas_ref, nad_ref,
             r2_ref, z2_ref, asn_ref, adn_ref, sex2_ref):
        sx = sex_ref[...]
        den = d0_ref[...] + d1_ref[...] + sx + 1e-16
        f = (num_ref[...] + sx * z_ref[...]) / den + cb_ref[...]
        y = f + r_ref[...]
        mu = jnp.mean(y, axis=1, keepdims=True)
        var = jnp.mean((y - mu) ** 2, axis=1, keepdims=True)
        xn = (y - mu) / jnp.sqrt(var + 1e-5) * g_ref[...] + b_ref[...]
        z2 = jnp.dot(xn, nw_ref[...], preferred_element_type=jnp.float32)
        a_s = jnp.dot(z2, nas_ref[...], preferred_element_type=jnp.float32)
        a_d = jnp.dot(z2, nad_ref[...], preferred_element_type=jnp.float32)
        r2_ref[...] = xn
        z2_ref[...] = z2
        asn_ref[...] = a_s
        adn_ref[...] = a_d
        sex2_ref[...] = jnp.exp(_lrelu(a_s + a_d))

    return pl.pallas_call(
        body,
        grid=(GN,),
        in_specs=[_node_spec(HID), _node_spec(1), _node_spec(1), _node_spec(1),
                  _node_spec(HID), _node_spec(HID)]
                 + [_full_spec(a.shape) for a in (cb, g, beta, nw, nas, nad)],
        out_specs=[_node_spec(HID), _node_spec(HID),
                   _node_spec(1), _node_spec(1), _node_spec(1)],
        out_shape=[jax.ShapeDtypeStruct((NP, HID), jnp.float32),
                   jax.ShapeDtypeStruct((NP, HID), jnp.float32),
                   jax.ShapeDtypeStruct((NP, 1), jnp.float32),
                   jax.ShapeDtypeStruct((NP, 1), jnp.float32),
                   jax.ShapeDtypeStruct((NP, 1), jnp.float32)],
    )(num, den0, den1, sex, z, r, cb, g, beta, nw, nas, nad)


def _tc_post(num, den0, den1, sex, z, r, cb, g, beta, ow1, ob1, ow2, ob2, ow3, ob3):
    def body(num_ref, d0_ref, d1_ref, sex_ref, z_ref, r_ref, cb_ref, g_ref,
             b_ref, w1r, b1r, w2r, b2r, w3r, b3r, o_ref):
        sx = sex_ref[...]
        den = d0_ref[...] + d1_ref[...] + sx + 1e-16
        f = (num_ref[...] + sx * z_ref[...]) / den + cb_ref[...]
        y = f + r_ref[...]
        mu = jnp.mean(y, axis=1, keepdims=True)
        var = jnp.mean((y - mu) ** 2, axis=1, keepdims=True)
        xn = (y - mu) / jnp.sqrt(var + 1e-5) * g_ref[...] + b_ref[...]
        o = _elu(jnp.dot(xn, w1r[...], preferred_element_type=jnp.float32) + b1r[...])
        o = _elu(jnp.dot(o, w2r[...], preferred_element_type=jnp.float32) + b2r[...])
        o_ref[...] = jnp.dot(o, w3r[...], preferred_element_type=jnp.float32) + b3r[...]

    return pl.pallas_call(
        body,
        grid=(GN,),
        in_specs=[_node_spec(HID), _node_spec(1), _node_spec(1), _node_spec(1),
                  _node_spec(HID), _node_spec(HID)]
                 + [_full_spec(a.shape)
                    for a in (cb, g, beta, ow1, ob1, ow2, ob2, ow3, ob3)],
        out_specs=[_node_spec(8)],
        out_shape=[jax.ShapeDtypeStruct((NP, 8), jnp.float32)],
    )(num, den0, den1, sex, z, r, cb, g, beta, ow1, ob1, ow2, ob2, ow3, ob3)[0]


def _sc_edge_ex(src, dst, asn, adn, kw):
    ep = src.shape[0]
    mesh = plsc.VectorSubcoreMesh(core_axis_name="c", subcore_axis_name="s",
                                  num_cores=NC, num_subcores=NS)

    @functools.partial(
        pl.kernel,
        out_type=[jax.ShapeDtypeStruct((ep,), jnp.float32),
                  jax.ShapeDtypeStruct((NC, NP), jnp.float32)],
        mesh=mesh,
        compiler_params=pltpu.CompilerParams(needs_layout_passes=False),
        scratch_types=[
            pltpu.VMEM((NP,), jnp.float32),
            pltpu.VMEM((CH,), jnp.int32),
            pltpu.VMEM((CH,), jnp.int32),
            pltpu.VMEM((NB, 128), jnp.int32),
            pltpu.VMEM((CH,), jnp.float32),
            pltpu.VMEM((CH,), jnp.float32),
            pltpu.VMEM((TS,), jnp.float32),
            pltpu.VMEM_SHARED((NP,), jnp.float32),
            pltpu.SemaphoreType.DMA,
        ],
    )
    def k(src_hbm, dst_hbm, asn_hbm, adn_hbm, ex_hbm, den_hbm,
          asn_v, srcl, dstl, dst2d, adv, exl, zb, den_sh, sem):
        c = lax.axis_index("c")
        s = lax.axis_index("s")
        w = s * NC + c

        def zi(i, _):
            zb[pl.ds(i * L, L)] = jnp.zeros((L,), jnp.float32)
            return 0
        lax.fori_loop(0, TS // L, zi, 0)
        pltpu.sync_copy(zb, den_sh.at[pl.ds(s * TS, TS)])
        pltpu.sync_copy(asn_hbm, asn_v)
        plsc.subcore_barrier()

        def chunk(i, _):
            c0 = (w * kw + i) * CH
            pltpu.sync_copy(src_hbm.at[pl.ds(c0, CH)], srcl)
            pltpu.sync_copy(dst_hbm.at[pl.ds(c0, CH)], dstl)

            def stage(b, _):
                pltpu.sync_copy(dst_hbm.at[pl.ds(c0 + b * 128, 128)], dst2d.at[b])
                pltpu.async_copy(adn_hbm.at[dstl.at[pl.ds(b * 128, 128)]],
                                 adv.at[pl.ds(b * 128, 128)], sem).wait()
                return 0
            lax.fori_loop(0, NB, stage, 0)

            def grp(gi, _):
                s16 = srcl[pl.ds(gi * L, L)]
                a_s = plsc.load_gather(asn_v, [s16])
                e16 = a_s + adv[pl.ds(gi * L, L)]
                e16 = jnp.where(e16 >= 0, e16, 0.2 * e16)
                exl[pl.ds(gi * L, L)] = jnp.exp(e16)
                return 0
            lax.fori_loop(0, CH // L, grp, 0)

            pltpu.sync_copy(exl, ex_hbm.at[pl.ds(c0, CH)])

            def dadd(b, _):
                pltpu.sync_copy(exl.at[pl.ds(b * 128, 128)],
                                den_sh.at[dst2d.at[b]], add=True)
                return 0
            lax.fori_loop(0, NB, dadd, 0)
            return 0
        lax.fori_loop(0, kw, chunk, 0)

        plsc.subcore_barrier()
        pltpu.sync_copy(den_sh.at[pl.ds(s * TS, TS)],
                        den_hbm.at[c, pl.ds(s * TS, TS)])

    return k(src, dst, asn, adn)


def _sc_aggregate(src, dst, ex, z, zrows):
    ep = src.shape[0]
    ncht = ep // CH2 // NS
    mesh = plsc.VectorSubcoreMesh(core_axis_name="c", subcore_axis_name="s",
                                  num_cores=NC, num_subcores=NS)

    @functools.partial(
        pl.kernel,
        out_type=jax.ShapeDtypeStruct((NP, HID), jnp.float32),
        mesh=mesh,
        compiler_params=pltpu.CompilerParams(needs_layout_passes=False,
                                             use_tc_tiling_on_sc=False),
        scratch_types=[
            pltpu.VMEM((CH2,), jnp.int32),
            pltpu.VMEM((CH2,), jnp.int32),
            pltpu.VMEM((CH2,), jnp.float32),
            pltpu.VMEM((PSZ,), jnp.int32),
            pltpu.VMEM((PSZ,), jnp.int32),
            pltpu.VMEM((PSZ,), jnp.float32),
            pltpu.VMEM((FB // 128, 128), jnp.int32),
            pltpu.VMEM((FB // 128, 128), jnp.int32),
            pltpu.VMEM((FB,), jnp.float32),
            pltpu.VMEM((FB, HID), jnp.float32),
            pltpu.VMEM((16, HID), jnp.float32),
            pltpu.VMEM_SHARED((R, HID), jnp.float32),
            pltpu.SemaphoreType.DMA,
        ],
    )
    def k(src_hbm, dst_hbm, ex_hbm, z_hbm, zr_hbm, num_hbm,
          srcl, dstl, exl, psrc, poff, pex, sidx, obat, wbat, rows, zb,
          acc, sem):
        c = lax.axis_index("c")
        s = lax.axis_index("s")
        pltpu.sync_copy(zr_hbm, zb)

        def fire(qbase, limit):
            for h in range(FB // 128):
                def bg(k_, _):
                    pos = qbase + h * 128 + k_ * L
                    lid = pos + lax.iota(jnp.int32, L)
                    valid = lid < limit
                    spread = (h * 128 + k_ * L) + lax.iota(jnp.int32, L)
                    sidx.at[h][pl.ds(k_ * L, L)] = jnp.where(
                        valid, psrc[pl.ds(pos, L)], spread)
                    obat.at[h][pl.ds(k_ * L, L)] = jnp.where(
                        valid, poff[pl.ds(pos, L)], spread)
                    wbat[pl.ds(h * 128 + k_ * L, L)] = jnp.where(
                        valid, pex[pl.ds(pos, L)], 0.0)
                    return 0
                lax.fori_loop(0, 128 // L, bg, 0)
            gds = [pltpu.async_copy(z_hbm.at[sidx.at[h]],
                                    rows.at[pl.ds(h * 128, 128)], sem)
                   for h in range(FB // 128)]
            for gd in gds:
                gd.wait()

            def scale(j, _):
                w16 = wbat[pl.ds(j * L, L)]
                e16 = j * L + lax.iota(jnp.int32, L)
                for cc in range(HID):
                    ci = jnp.full((L,), cc, jnp.int32)
                    v = plsc.load_gather(rows, [e16, ci])
                    plsc.store_scatter(rows, [e16, ci], v * w16)
                return 0
            lax.fori_loop(0, FB // L, scale, 0)
            for h in range(FB // 128):
                pltpu.sync_copy(rows.at[pl.ds(h * 128, 128)],
                                acc.at[obat.at[h]], add=True)

        def rngloop(rng, _):
            g = c * (NR // NC) + rng
            base = g * R

            def zi(i, _):
                pltpu.sync_copy(zb, acc.at[pl.ds(s * (R // NS) + i * 16, 16)])
                return 0
            lax.fori_loop(0, R // NS // 16, zi, 0)
            plsc.subcore_barrier()

            def chunk(i, pcount):
                c0 = (s * ncht + i) * CH2
                pltpu.sync_copy(src_hbm.at[pl.ds(c0, CH2)], srcl)
                pltpu.sync_copy(dst_hbm.at[pl.ds(c0, CH2)], dstl)
                pltpu.sync_copy(ex_hbm.at[pl.ds(c0, CH2)], exl)

                def cmp(j, p):
                    d16 = dstl[pl.ds(j * L, L)]
                    inr = (d16 >= base) & (d16 < base + R)
                    plsc.store_compressed(psrc.at[pl.ds(p, L)],
                                          srcl[pl.ds(j * L, L)], mask=inr)
                    plsc.store_compressed(poff.at[pl.ds(p, L)],
                                          d16 - base, mask=inr)
                    plsc.store_compressed(pex.at[pl.ds(p, L)],
                                          exl[pl.ds(j * L, L)], mask=inr)
                    return p + jnp.sum(inr.astype(jnp.int32))
                p1 = lax.fori_loop(0, CH2 // L, cmp, pcount)

                def drain_cond(q):
                    return p1 - q >= FB

                def drain(q):
                    fire(q, p1)
                    return q + FB
                q1 = lax.while_loop(drain_cond, drain, jnp.int32(0))

                def mv(k_, _):
                    psrc[pl.ds(k_ * L, L)] = psrc[pl.ds(q1 + k_ * L, L)]
                    poff[pl.ds(k_ * L, L)] = poff[pl.ds(q1 + k_ * L, L)]
                    pex[pl.ds(k_ * L, L)] = pex[pl.ds(q1 + k_ * L, L)]
                    return 0
                lax.fori_loop(0, FB // L, mv, 0)
                return p1 - q1
            pfin = lax.fori_loop(0, ncht, chunk, jnp.int32(0))
            fire(jnp.int32(0), pfin)

            plsc.subcore_barrier()
            pltpu.sync_copy(acc.at[pl.ds(s * (R // NS), R // NS)],
                            num_hbm.at[pl.ds(base + s * (R // NS), R // NS)])
            plsc.subcore_barrier()
            return 0
        lax.fori_loop(0, NR // NC, rngloop, 0)

    return k(src, dst, ex, z, zrows)


def kernel(x_lc, edge_index, batch_lc, enc_W1, enc_b1, enc_W2, enc_b2, enc_W3,
           enc_b3, conv1_W, conv1_as, conv1_ad, conv1_b, norm1_g, norm1_beta,
           conv2_W, conv2_as, conv2_ad, conv2_b, norm2_g, norm2_beta, conv3_W,
           conv3_as, conv3_ad, conv3_b, norm3_g, norm3_beta, out_W1, out_b1,
           out_W2, out_b2, out_W3, out_b3):
    n = x_lc.shape[0]
    e = edge_index.shape[1]
    kw = -(-e // (NW * CH))
    ep = NW * kw * CH

    src = edge_index[0]
    dst = edge_index[1]
    pid = jnp.arange(ep - e, dtype=jnp.int32)
    src_p = jnp.concatenate([src, pid % 1024])
    dst_p = jnp.concatenate([dst, n + (pid % 64)])

    xp = jnp.pad(x_lc, ((0, NP - n), (0, 1)))
    w1 = jnp.pad(enc_W1, ((0, 1), (0, 0)))
    row = lambda v: v.reshape(1, -1)
    col = lambda v: v.reshape(-1, 1)
    zrows = jnp.zeros((16, HID), jnp.float32)

    r, z, asn, adn, sex = _tc_encode(
        xp, w1, row(enc_b1), enc_W2, row(enc_b2), enc_W3, row(enc_b3),
        conv1_W, col(conv1_as), col(conv1_ad))

    layers = [
        (conv1_b, norm1_g, norm1_beta, conv2_W, conv2_as, conv2_ad),
        (conv2_b, norm2_g, norm2_beta, conv3_W, conv3_as, conv3_ad),
    ]
    for cb, g, beta, nw_, nas, nad in layers:
        ex, den = _sc_edge_ex(src_p, dst_p, asn.reshape(NP), adn.reshape(NP), kw)
        num = _sc_aggregate(src_p, dst_p, ex, z, zrows)
        r, z, asn, adn, sex = _tc_mid(
            num, col(den[0]), col(den[1]), sex, z, r,
            row(cb), row(g), row(beta), nw_, col(nas), col(nad))

    ex, den = _sc_edge_ex(src_p, dst_p, asn.reshape(NP), adn.reshape(NP), kw)
    num = _sc_aggregate(src_p, dst_p, ex, z, zrows)
    o = _tc_post(num, col(den[0]), col(den[1]), sex, z, r,
                 row(conv3_b), row(norm3_g), row(norm3_beta),
                 out_W1, row(out_b1), out_W2, row(out_b2), out_W3, row(out_b3))

    return (o[:n], batch_lc)


# speedup vs baseline: 9.6468x; 1.0453x over previous
import functools

import jax
import jax.numpy as jnp
from jax import lax
from jax.experimental import pallas as pl
from jax.experimental.pallas import tpu as pltpu
from jax.experimental.pallas import tpu_sc as plsc

NC = 2
NS = 16
NW = NC * NS
L = 16

NP = 100352
NR = 4
R = NP // NR
CH2 = 1024
PSZ = CH2 + 128 + L
TS = NP // NS
CH = 1024
NB = CH // 128
HID = 64
BN = 512
GN = NP // BN


def _elu(x):
    return jnp.where(x > 0, x, jnp.exp(x) - 1.0)


def _lrelu(x):
    return jnp.where(x >= 0, x, 0.2 * x)


def _full_spec(shape):
    nd = len(shape)
    return pl.BlockSpec(shape, lambda i, _nd=nd: (0,) * _nd)


def _node_spec(cols):
    return pl.BlockSpec((BN, cols), lambda i: (i, 0))


def _tc_encode(xp, w1, b1, w2, b2, w3, b3, cw, cas, cad):
    def body(x_ref, w1r, b1r, w2r, b2r, w3r, b3r, cwr, casr, cadr,
             r_ref, z_ref, asn_ref, adn_ref, sex_ref):
        x = x_ref[...]
        h = _elu(jnp.dot(x, w1r[...], preferred_element_type=jnp.float32) + b1r[...])
        h = _elu(jnp.dot(h, w2r[...], preferred_element_type=jnp.float32) + b2r[...])
        h = jnp.dot(h, w3r[...], preferred_element_type=jnp.float32) + b3r[...]
        z = jnp.dot(h, cwr[...], preferred_element_type=jnp.float32)
        a_s = jnp.dot(z, casr[...], preferred_element_type=jnp.float32)
        a_d = jnp.dot(z, cadr[...], preferred_element_type=jnp.float32)
        r_ref[...] = h
        z_ref[...] = z
        asn_ref[...] = a_s
        adn_ref[...] = a_d
        sex_ref[...] = jnp.exp(_lrelu(a_s + a_d))

    return pl.pallas_call(
        body,
        grid=(GN,),
        in_specs=[_node_spec(16)] + [_full_spec(a.shape)
                                     for a in (w1, b1, w2, b2, w3, b3, cw, cas, cad)],
        out_specs=[_node_spec(HID), _node_spec(HID),
                   _node_spec(1), _node_spec(1), _node_spec(1)],
        out_shape=[jax.ShapeDtypeStruct((NP, HID), jnp.float32),
                   jax.ShapeDtypeStruct((NP, HID), jnp.float32),
                   jax.ShapeDtypeStruct((NP, 1), jnp.float32),
                   jax.ShapeDtypeStruct((NP, 1), jnp.float32),
                   jax.ShapeDtypeStruct((NP, 1), jnp.float32)],
    )(xp, w1, b1, w2, b2, w3, b3, cw, cas, cad)


def _tc_mid(num, den0, den1, sex, z, r, cb, g, beta, nw, nas, nad):
    def body(num_ref, d0_ref, d1_ref, sex_ref, z_ref, r_ref, cb_ref, g_ref,
             b_ref, nw_ref, nas_ref, nad_ref,
             r2_ref, z2_ref, asn_ref, adn_ref, sex2_ref):
        sx = sex_ref[...]
        den = d0_ref[...] + d1_ref[...] + sx + 1e-16
        f = (num_ref[...] + sx * z_ref[...]) / den + cb_ref[...]
        y = f + r_ref[...]
        mu = jnp.mean(y, axis=1, keepdims=True)
        var = jnp.mean((y - mu) ** 2, axis=1, keepdims=True)
        xn = (y - mu) / jnp.sqrt(var + 1e-5) * g_ref[...] + b_ref[...]
        z2 = jnp.dot(xn, nw_ref[...], preferred_element_type=jnp.float32)
        a_s = jnp.dot(z2, nas_ref[...], preferred_element_type=jnp.float32)
        a_d = jnp.dot(z2, nad_ref[...], preferred_element_type=jnp.float32)
        r2_ref[...] = xn
        z2_ref[...] = z2
        asn_ref[...] = a_s
        adn_ref[...] = a_d
        sex2_ref[...] = jnp.exp(_lrelu(a_s + a_d))

    return pl.pallas_call(
        body,
        grid=(GN,),
        in_specs=[_node_spec(HID), _node_spec(1), _node_spec(1), _node_spec(1),
                  _node_spec(HID), _node_spec(HID)]
                 + [_full_spec(a.shape) for a in (cb, g, beta, nw, nas, nad)],
        out_specs=[_node_spec(HID), _node_spec(HID),
                   _node_spec(1), _node_spec(1), _node_spec(1)],
        out_shape=[jax.ShapeDtypeStruct((NP, HID), jnp.float32),
                   jax.ShapeDtypeStruct((NP, HID), jnp.float32),
                   jax.ShapeDtypeStruct((NP, 1), jnp.float32),
                   jax.ShapeDtypeStruct((NP, 1), jnp.float32),
                   jax.ShapeDtypeStruct((NP, 1), jnp.float32)],
    )(num, den0, den1, sex, z, r, cb, g, beta, nw, nas, nad)


def _tc_post(num, den0, den1, sex, z, r, cb, g, beta, ow1, ob1, ow2, ob2, ow3, ob3):
    def body(num_ref, d0_ref, d1_ref, sex_ref, z_ref, r_ref, cb_ref, g_ref,
             b_ref, w1r, b1r, w2r, b2r, w3r, b3r, o_ref):
        sx = sex_ref[...]
        den = d0_ref[...] + d1_ref[...] + sx + 1e-16
        f = (num_ref[...] + sx * z_ref[...]) / den + cb_ref[...]
        y = f + r_ref[...]
        mu = jnp.mean(y, axis=1, keepdims=True)
        var = jnp.mean((y - mu) ** 2, axis=1, keepdims=True)
        xn = (y - mu) / jnp.sqrt(var + 1e-5) * g_ref[...] + b_ref[...]
        o = _elu(jnp.dot(xn, w1r[...], preferred_element_type=jnp.float32) + b1r[...])
        o = _elu(jnp.dot(o, w2r[...], preferred_element_type=jnp.float32) + b2r[...])
        o_ref[...] = jnp.dot(o, w3r[...], preferred_element_type=jnp.float32) + b3r[...]

    return pl.pallas_call(
        body,
        grid=(GN,),
        in_specs=[_node_spec(HID), _node_spec(1), _node_spec(1), _node_spec(1),
                  _node_spec(HID), _node_spec(HID)]
                 + [_full_spec(a.shape)
                    for a in (cb, g, beta, ow1, ob1, ow2, ob2, ow3, ob3)],
        out_specs=[_node_spec(8)],
        out_shape=[jax.ShapeDtypeStruct((NP, 8), jnp.float32)],
    )(num, den0, den1, sex, z, r, cb, g, beta, ow1, ob1, ow2, ob2, ow3, ob3)[0]


def _sc_edge_ex(src, dst, asn, adn, kw):
    ep = src.shape[0]
    mesh = plsc.VectorSubcoreMesh(core_axis_name="c", subcore_axis_name="s",
                                  num_cores=NC, num_subcores=NS)

    @functools.partial(
        pl.kernel,
        out_type=[jax.ShapeDtypeStruct((ep,), jnp.float32),
                  jax.ShapeDtypeStruct((NC, NP), jnp.float32)],
        mesh=mesh,
        compiler_params=pltpu.CompilerParams(needs_layout_passes=False),
        scratch_types=[
            pltpu.VMEM((NP,), jnp.float32),
            pltpu.VMEM((CH,), jnp.int32),
            pltpu.VMEM((CH,), jnp.int32),
            pltpu.VMEM((NB, 128), jnp.int32),
            pltpu.VMEM((CH,), jnp.float32),
            pltpu.VMEM((CH,), jnp.float32),
            pltpu.VMEM((TS,), jnp.float32),
            pltpu.VMEM_SHARED((NP,), jnp.float32),
            pltpu.SemaphoreType.DMA,
        ],
    )
    def k(src_hbm, dst_hbm, asn_hbm, adn_hbm, ex_hbm, den_hbm,
          asn_v, srcl, dstl, dst2d, adv, exl, zb, den_sh, sem):
        c = lax.axis_index("c")
        s = lax.axis_index("s")
        w = s * NC + c

        def zi(i, _):
            zb[pl.ds(i * L, L)] = jnp.zeros((L,), jnp.float32)
            return 0
        lax.fori_loop(0, TS // L, zi, 0)
        pltpu.sync_copy(zb, den_sh.at[pl.ds(s * TS, TS)])
        pltpu.sync_copy(asn_hbm, asn_v)
        plsc.subcore_barrier()

        def chunk(i, _):
            c0 = (w * kw + i) * CH
            pltpu.sync_copy(src_hbm.at[pl.ds(c0, CH)], srcl)
            pltpu.sync_copy(dst_hbm.at[pl.ds(c0, CH)], dstl)

            def stage(b, _):
                pltpu.sync_copy(dst_hbm.at[pl.ds(c0 + b * 128, 128)], dst2d.at[b])
                pltpu.async_copy(adn_hbm.at[dstl.at[pl.ds(b * 128, 128)]],
                                 adv.at[pl.ds(b * 128, 128)], sem).wait()
                return 0
            lax.fori_loop(0, NB, stage, 0)

            def grp(gi, _):
                s16 = srcl[pl.ds(gi * L, L)]
                a_s = plsc.load_gather(asn_v, [s16])
                e16 = a_s + adv[pl.ds(gi * L, L)]
                e16 = jnp.where(e16 >= 0, e16, 0.2 * e16)
                exl[pl.ds(gi * L, L)] = jnp.exp(e16)
                return 0
            lax.fori_loop(0, CH // L, grp, 0)

            pltpu.sync_copy(exl, ex_hbm.at[pl.ds(c0, CH)])

            def dadd(b, _):
                pltpu.sync_copy(exl.at[pl.ds(b * 128, 128)],
                                den_sh.at[dst2d.at[b]], add=True)
                return 0
            lax.fori_loop(0, NB, dadd, 0)
            return 0
        lax.fori_loop(0, kw, chunk, 0)

        plsc.subcore_barrier()
        pltpu.sync_copy(den_sh.at[pl.ds(s * TS, TS)],
                        den_hbm.at[c, pl.ds(s * TS, TS)])

    return k(src, dst, asn, adn)


def _sc_aggregate(src, dst, ex, z, zrows):
    ep = src.shape[0]
    ncht = ep // CH2 // NS
    mesh = plsc.VectorSubcoreMesh(core_axis_name="c", subcore_axis_name="s",
                                  num_cores=NC, num_subcores=NS)

    @functools.partial(
        pl.kernel,
        out_type=jax.ShapeDtypeStruct((NP, HID), jnp.float32),
        mesh=mesh,
        compiler_params=pltpu.CompilerParams(needs_layout_passes=False,
                                             use_tc_tiling_on_sc=False),
        scratch_types=[
            pltpu.VMEM((CH2,), jnp.int32),
            pltpu.VMEM((CH2,), jnp.int32),
            pltpu.VMEM((CH2,), jnp.float32),
            pltpu.VMEM((PSZ,), jnp.int32),
            pltpu.VMEM((PSZ,), jnp.int32),
            pltpu.VMEM((PSZ,), jnp.float32),
            pltpu.VMEM((2, 128), jnp.int32),
            pltpu.VMEM((2, 128), jnp.int32),
            pltpu.VMEM((2, 128), jnp.float32),
            pltpu.VMEM((2, 128, HID), jnp.float32),
            pltpu.VMEM((16, HID), jnp.float32),
            pltpu.VMEM_SHARED((R, HID), jnp.float32),
            pltpu.SemaphoreType.DMA,
        ],
    )
    def k(src_hbm, dst_hbm, ex_hbm, z_hbm, zr_hbm, num_hbm,
          srcl, dstl, exl, psrc, poff, pex, sidx, obat, wbat, rows, zb,
          acc, sem):
        c = lax.axis_index("c")
        s = lax.axis_index("s")
        pltpu.sync_copy(zr_hbm, zb)

        def proc(par):
            pltpu.make_async_copy(z_hbm.at[pl.ds(0, 128)],
                                  rows.at[par], sem).wait()

            def scale(j, _):
                w16 = wbat[par, pl.ds(j * L, L)]
                e16 = j * L + lax.iota(jnp.int32, L)
                for cc in range(HID):
                    ci = jnp.full((L,), cc, jnp.int32)
                    v = plsc.load_gather(rows.at[par], [e16, ci])
                    plsc.store_scatter(rows.at[par], [e16, ci], v * w16)
                return 0
            lax.fori_loop(0, 128 // L, scale, 0)
            pltpu.sync_copy(rows.at[par], acc.at[obat.at[par]], add=True)

        def fire(qbase, limit, nb):
            par = nb % 2

            def bg(k_, _):
                pos = qbase + k_ * L
                lid = pos + lax.iota(jnp.int32, L)
                valid = lid < limit
                spread = (k_ * L) + lax.iota(jnp.int32, L)
                sidx.at[par][pl.ds(k_ * L, L)] = jnp.where(
                    valid, psrc[pl.ds(pos, L)], spread)
                obat.at[par][pl.ds(k_ * L, L)] = jnp.where(
                    valid, poff[pl.ds(pos, L)], spread)
                wbat.at[par][pl.ds(k_ * L, L)] = jnp.where(
                    valid, pex[pl.ds(pos, L)], 0.0)
                return 0
            lax.fori_loop(0, 128 // L, bg, 0)
            pltpu.async_copy(z_hbm.at[sidx.at[par]], rows.at[par], sem)

            @pl.when(nb >= 1)
            def _():
                proc((nb - 1) % 2)
            return nb + 1

        def rngloop(rng, _):
            g = c * (NR // NC) + rng
            base = g * R

            def zi(i, _):
                pltpu.sync_copy(zb, acc.at[pl.ds(s * (R // NS) + i * 16, 16)])
                return 0
            lax.fori_loop(0, R // NS // 16, zi, 0)
            plsc.subcore_barrier()

            def chunk(i, st):
                pcount, nb0 = st
                c0 = (s * ncht + i) * CH2
                pltpu.sync_copy(src_hbm.at[pl.ds(c0, CH2)], srcl)
                pltpu.sync_copy(dst_hbm.at[pl.ds(c0, CH2)], dstl)
                pltpu.sync_copy(ex_hbm.at[pl.ds(c0, CH2)], exl)

                def cmp(j, p):
                    d16 = dstl[pl.ds(j * L, L)]
                    inr = (d16 >= base) & (d16 < base + R)
                    plsc.store_compressed(psrc.at[pl.ds(p, L)],
                                          srcl[pl.ds(j * L, L)], mask=inr)
                    plsc.store_compressed(poff.at[pl.ds(p, L)],
                                          d16 - base, mask=inr)
                    plsc.store_compressed(pex.at[pl.ds(p, L)],
                                          exl[pl.ds(j * L, L)], mask=inr)
                    return p + jnp.sum(inr.astype(jnp.int32))
                p1 = lax.fori_loop(0, CH2 // L, cmp, pcount)

                def drain_cond(qn):
                    return p1 - qn[0] >= 128

                def drain(qn):
                    q, nb = qn
                    nb2 = fire(q, p1, nb)
                    return (q + 128, nb2)
                q1, nb1 = lax.while_loop(drain_cond, drain,
                                         (jnp.int32(0), nb0))

                def mv(k_, _):
                    psrc[pl.ds(k_ * L, L)] = psrc[pl.ds(q1 + k_ * L, L)]
                    poff[pl.ds(k_ * L, L)] = poff[pl.ds(q1 + k_ * L, L)]
                    pex[pl.ds(k_ * L, L)] = pex[pl.ds(q1 + k_ * L, L)]
                    return 0
                lax.fori_loop(0, 128 // L, mv, 0)
                return (p1 - q1, nb1)
            pfin, nbf = lax.fori_loop(0, ncht, chunk,
                                      (jnp.int32(0), jnp.int32(0)))
            nbf2 = fire(jnp.int32(0), pfin, nbf)
            proc((nbf2 - 1) % 2)

            plsc.subcore_barrier()
            pltpu.sync_copy(acc.at[pl.ds(s * (R // NS), R // NS)],
                            num_hbm.at[pl.ds(base + s * (R // NS), R // NS)])
            plsc.subcore_barrier()
            return 0
        lax.fori_loop(0, NR // NC, rngloop, 0)

    return k(src, dst, ex, z, zrows)


def kernel(x_lc, edge_index, batch_lc, enc_W1, enc_b1, enc_W2, enc_b2, enc_W3,
           enc_b3, conv1_W, conv1_as, conv1_ad, conv1_b, norm1_g, norm1_beta,
           conv2_W, conv2_as, conv2_ad, conv2_b, norm2_g, norm2_beta, conv3_W,
           conv3_as, conv3_ad, conv3_b, norm3_g, norm3_beta, out_W1, out_b1,
           out_W2, out_b2, out_W3, out_b3):
    n = x_lc.shape[0]
    e = edge_index.shape[1]
    kw = -(-e // (NW * CH))
    ep = NW * kw * CH

    src = edge_index[0]
    dst = edge_index[1]
    pid = jnp.arange(ep - e, dtype=jnp.int32)
    src_p = jnp.concatenate([src, pid % 1024])
    dst_p = jnp.concatenate([dst, n + (pid % 64)])

    xp = jnp.pad(x_lc, ((0, NP - n), (0, 1)))
    w1 = jnp.pad(enc_W1, ((0, 1), (0, 0)))
    row = lambda v: v.reshape(1, -1)
    col = lambda v: v.reshape(-1, 1)
    zrows = jnp.zeros((16, HID), jnp.float32)

    r, z, asn, adn, sex = _tc_encode(
        xp, w1, row(enc_b1), enc_W2, row(enc_b2), enc_W3, row(enc_b3),
        conv1_W, col(conv1_as), col(conv1_ad))

    layers = [
        (conv1_b, norm1_g, norm1_beta, conv2_W, conv2_as, conv2_ad),
        (conv2_b, norm2_g, norm2_beta, conv3_W, conv3_as, conv3_ad),
    ]
    for cb, g, beta, nw_, nas, nad in layers:
        ex, den = _sc_edge_ex(src_p, dst_p, asn.reshape(NP), adn.reshape(NP), kw)
        num = _sc_aggregate(src_p, dst_p, ex, z, zrows)
        r, z, asn, adn, sex = _tc_mid(
            num, col(den[0]), col(den[1]), sex, z, r,
            row(cb), row(g), row(beta), nw_, col(nas), col(nad))

    ex, den = _sc_edge_ex(src_p, dst_p, asn.reshape(NP), adn.reshape(NP), kw)
    num = _sc_aggregate(src_p, dst_p, ex, z, zrows)
    o = _tc_post(num, col(den[0]), col(den[1]), sex, z, r,
                 row(conv3_b), row(norm3_g), row(norm3_beta),
                 out_W1, row(out_b1), out_W2, row(out_b2), out_W3, row(out_b3))

    return (o[:n], batch_lc)


# speedup vs baseline: 11.6627x; 1.2090x over previous
import functools

import jax
import jax.numpy as jnp
from jax import lax
from jax.experimental import pallas as pl
from jax.experimental.pallas import tpu as pltpu
from jax.experimental.pallas import tpu_sc as plsc

NC = 2
NS = 16
NW = NC * NS
L = 16

NP = 100352
NR = 4
R = NP // NR
CH2 = 1024
PSZ = CH2 + 128 + L
TS = NP // NS
CH = 1024
NB = CH // 128
HID = 64
BN = 512
GN = NP // BN


def _elu(x):
    return jnp.where(x > 0, x, jnp.exp(x) - 1.0)


def _lrelu(x):
    return jnp.where(x >= 0, x, 0.2 * x)


def _full_spec(shape):
    nd = len(shape)
    return pl.BlockSpec(shape, lambda i, _nd=nd: (0,) * _nd)


def _node_spec(cols):
    return pl.BlockSpec((BN, cols), lambda i: (i, 0))


def _tc_encode(xp, w1, b1, w2, b2, w3, b3, cw, cas, cad):
    def body(x_ref, w1r, b1r, w2r, b2r, w3r, b3r, cwr, casr, cadr,
             r_ref, z_ref, asn_ref, adn_ref, sex_ref):
        x = x_ref[...]
        h = _elu(jnp.dot(x, w1r[...], preferred_element_type=jnp.float32) + b1r[...])
        h = _elu(jnp.dot(h, w2r[...], preferred_element_type=jnp.float32) + b2r[...])
        h = jnp.dot(h, w3r[...], preferred_element_type=jnp.float32) + b3r[...]
        z = jnp.dot(h, cwr[...], preferred_element_type=jnp.float32)
        a_s = jnp.dot(z, casr[...], preferred_element_type=jnp.float32)
        a_d = jnp.dot(z, cadr[...], preferred_element_type=jnp.float32)
        r_ref[...] = h
        z_ref[...] = z
        asn_ref[...] = a_s
        adn_ref[...] = a_d
        sex_ref[...] = jnp.exp(_lrelu(a_s + a_d))

    return pl.pallas_call(
        body,
        grid=(GN,),
        in_specs=[_node_spec(16)] + [_full_spec(a.shape)
                                     for a in (w1, b1, w2, b2, w3, b3, cw, cas, cad)],
        out_specs=[_node_spec(HID), _node_spec(HID),
                   _node_spec(1), _node_spec(1), _node_spec(1)],
        out_shape=[jax.ShapeDtypeStruct((NP, HID), jnp.float32),
                   jax.ShapeDtypeStruct((NP, HID), jnp.float32),
                   jax.ShapeDtypeStruct((NP, 1), jnp.float32),
                   jax.ShapeDtypeStruct((NP, 1), jnp.float32),
                   jax.ShapeDtypeStruct((NP, 1), jnp.float32)],
    )(xp, w1, b1, w2, b2, w3, b3, cw, cas, cad)


def _tc_mid(num, den0, den1, sex, z, r, cb, g, beta, nw, nas, nad):
    def body(num_ref, d0_ref, d1_ref, sex_ref, z_ref, r_ref, cb_ref, g_ref,
             b_ref, nw_ref, nas_ref, nad_ref,
             r2_ref, z2_ref, asn_ref, adn_ref, sex2_ref):
        sx = sex_ref[...]
        den = d0_ref[...] + d1_ref[...] + sx + 1e-16
        f = (num_ref[...] + sx * z_ref[...]) / den + cb_ref[...]
        y = f + r_ref[...]
        mu = jnp.mean(y, axis=1, keepdims=True)
        var = jnp.mean((y - mu) ** 2, axis=1, keepdims=True)
        xn = (y - mu) / jnp.sqrt(var + 1e-5) * g_ref[...] + b_ref[...]
        z2 = jnp.dot(xn, nw_ref[...], preferred_element_type=jnp.float32)
        a_s = jnp.dot(z2, nas_ref[...], preferred_element_type=jnp.float32)
        a_d = jnp.dot(z2, nad_ref[...], preferred_element_type=jnp.float32)
        r2_ref[...] = xn
        z2_ref[...] = z2
        asn_ref[...] = a_s
        adn_ref[...] = a_d
        sex2_ref[...] = jnp.exp(_lrelu(a_s + a_d))

    return pl.pallas_call(
        body,
        grid=(GN,),
        in_specs=[_node_spec(HID), _node_spec(1), _node_spec(1), _node_spec(1),
                  _node_spec(HID), _node_spec(HID)]
                 + [_full_spec(a.shape) for a in (cb, g, beta, nw, nas, nad)],
        out_specs=[_node_spec(HID), _node_spec(HID),
                   _node_spec(1), _node_spec(1), _node_spec(1)],
        out_shape=[jax.ShapeDtypeStruct((NP, HID), jnp.float32),
                   jax.ShapeDtypeStruct((NP, HID), jnp.float32),
                   jax.ShapeDtypeStruct((NP, 1), jnp.float32),
                   jax.ShapeDtypeStruct((NP, 1), jnp.float32),
                   jax.ShapeDtypeStruct((NP, 1), jnp.float32)],
    )(num, den0, den1, sex, z, r, cb, g, beta, nw, nas, nad)


def _tc_post(num, den0, den1, sex, z, r, cb, g, beta, ow1, ob1, ow2, ob2, ow3, ob3):
    def body(num_ref, d0_ref, d1_ref, sex_ref, z_ref, r_ref, cb_ref, g_ref,
             b_ref, w1r, b1r, w2r, b2r, w3r, b3r, o_ref):
        sx = sex_ref[...]
        den = d0_ref[...] + d1_ref[...] + sx + 1e-16
        f = (num_ref[...] + sx * z_ref[...]) / den + cb_ref[...]
        y = f + r_ref[...]
        mu = jnp.mean(y, axis=1, keepdims=True)
        var = jnp.mean((y - mu) ** 2, axis=1, keepdims=True)
        xn = (y - mu) / jnp.sqrt(var + 1e-5) * g_ref[...] + b_ref[...]
        o = _elu(jnp.dot(xn, w1r[...], preferred_element_type=jnp.float32) + b1r[...])
        o = _elu(jnp.dot(o, w2r[...], preferred_element_type=jnp.float32) + b2r[...])
        o_ref[...] = jnp.dot(o, w3r[...], preferred_element_type=jnp.float32) + b3r[...]

    return pl.pallas_call(
        body,
        grid=(GN,),
        in_specs=[_node_spec(HID), _node_spec(1), _node_spec(1), _node_spec(1),
                  _node_spec(HID), _node_spec(HID)]
                 + [_full_spec(a.shape)
                    for a in (cb, g, beta, ow1, ob1, ow2, ob2, ow3, ob3)],
        out_specs=[_node_spec(8)],
        out_shape=[jax.ShapeDtypeStruct((NP, 8), jnp.float32)],
    )(num, den0, den1, sex, z, r, cb, g, beta, ow1, ob1, ow2, ob2, ow3, ob3)[0]


def _sc_edge_ex(src, dst, asn, adn, kw):
    ep = src.shape[0]
    mesh = plsc.VectorSubcoreMesh(core_axis_name="c", subcore_axis_name="s",
                                  num_cores=NC, num_subcores=NS)

    @functools.partial(
        pl.kernel,
        out_type=[jax.ShapeDtypeStruct((ep,), jnp.float32),
                  jax.ShapeDtypeStruct((NC, NP), jnp.float32)],
        mesh=mesh,
        compiler_params=pltpu.CompilerParams(needs_layout_passes=False),
        scratch_types=[
            pltpu.VMEM((NP,), jnp.float32),
            pltpu.VMEM((CH,), jnp.int32),
            pltpu.VMEM((CH,), jnp.int32),
            pltpu.VMEM((NB, 128), jnp.int32),
            pltpu.VMEM((CH,), jnp.float32),
            pltpu.VMEM((CH,), jnp.float32),
            pltpu.VMEM((TS,), jnp.float32),
            pltpu.VMEM_SHARED((NP,), jnp.float32),
            pltpu.SemaphoreType.DMA,
        ],
    )
    def k(src_hbm, dst_hbm, asn_hbm, adn_hbm, ex_hbm, den_hbm,
          asn_v, srcl, dstl, dst2d, adv, exl, zb, den_sh, sem):
        c = lax.axis_index("c")
        s = lax.axis_index("s")
        w = s * NC + c

        def zi(i, _):
            zb[pl.ds(i * L, L)] = jnp.zeros((L,), jnp.float32)
            return 0
        lax.fori_loop(0, TS // L, zi, 0)
        pltpu.sync_copy(zb, den_sh.at[pl.ds(s * TS, TS)])
        pltpu.sync_copy(asn_hbm, asn_v)
        plsc.subcore_barrier()

        def chunk(i, _):
            c0 = (w * kw + i) * CH
            pltpu.sync_copy(src_hbm.at[pl.ds(c0, CH)], srcl)
            pltpu.sync_copy(dst_hbm.at[pl.ds(c0, CH)], dstl)

            def stage(b, _):
                pltpu.sync_copy(dst_hbm.at[pl.ds(c0 + b * 128, 128)], dst2d.at[b])
                pltpu.async_copy(adn_hbm.at[dstl.at[pl.ds(b * 128, 128)]],
                                 adv.at[pl.ds(b * 128, 128)], sem).wait()
                return 0
            lax.fori_loop(0, NB, stage, 0)

            def grp(gi, _):
                s16 = srcl[pl.ds(gi * L, L)]
                a_s = plsc.load_gather(asn_v, [s16])
                e16 = a_s + adv[pl.ds(gi * L, L)]
                e16 = jnp.where(e16 >= 0, e16, 0.2 * e16)
                exl[pl.ds(gi * L, L)] = jnp.exp(e16)
                return 0
            lax.fori_loop(0, CH // L, grp, 0)

            pltpu.sync_copy(exl, ex_hbm.at[pl.ds(c0, CH)])

            def dadd(b, _):
                pltpu.sync_copy(exl.at[pl.ds(b * 128, 128)],
                                den_sh.at[dst2d.at[b]], add=True)
                return 0
            lax.fori_loop(0, NB, dadd, 0)
            return 0
        lax.fori_loop(0, kw, chunk, 0)

        plsc.subcore_barrier()
        pltpu.sync_copy(den_sh.at[pl.ds(s * TS, TS)],
                        den_hbm.at[c, pl.ds(s * TS, TS)])

    return k(src, dst, asn, adn)


def _sc_aggregate(src, dst, ex, z, zrows):
    ep = src.shape[0]
    ncht = ep // CH2 // NS
    mesh = plsc.VectorSubcoreMesh(core_axis_name="c", subcore_axis_name="s",
                                  num_cores=NC, num_subcores=NS)

    @functools.partial(
        pl.kernel,
        out_type=jax.ShapeDtypeStruct((NP, HID), jnp.float32),
        mesh=mesh,
        compiler_params=pltpu.CompilerParams(needs_layout_passes=False,
                                             use_tc_tiling_on_sc=False),
        scratch_types=[
            pltpu.VMEM((CH2,), jnp.int32),
            pltpu.VMEM((CH2,), jnp.int32),
            pltpu.VMEM((CH2,), jnp.float32),
            pltpu.VMEM((PSZ,), jnp.int32),
            pltpu.VMEM((PSZ,), jnp.int32),
            pltpu.VMEM((PSZ,), jnp.float32),
            pltpu.VMEM((2, 128), jnp.int32),
            pltpu.VMEM((2, 128), jnp.int32),
            pltpu.VMEM((2, 128), jnp.float32),
            pltpu.VMEM((2, 128, HID // 2), jnp.int32),
            pltpu.VMEM((128, HID), jnp.float32),
            pltpu.VMEM((16, HID), jnp.float32),
            pltpu.VMEM_SHARED((R, HID), jnp.float32),
            pltpu.SemaphoreType.DMA,
        ],
    )
    def k(src_hbm, dst_hbm, ex_hbm, z_hbm, zr_hbm, num_hbm,
          srcl, dstl, exl, psrc, poff, pex, sidx, obat, wbat, rows, rowsf,
          zb, acc, sem):
        c = lax.axis_index("c")
        s = lax.axis_index("s")
        pltpu.sync_copy(zr_hbm, zb)

        def proc(par):
            pltpu.make_async_copy(z_hbm.at[pl.ds(0, 128)],
                                  rows.at[par], sem).wait()

            def scale(j, _):
                w16 = wbat[par, pl.ds(j * L, L)]
                e16 = j * L + lax.iota(jnp.int32, L)
                for cc in range(HID // 2):
                    ci = jnp.full((L,), cc, jnp.int32)
                    vi = plsc.load_gather(rows.at[par], [e16, ci])
                    vb = plsc.bitcast(vi, jnp.bfloat16)
                    a, b = plsc.unpack(vb, format=plsc.PackFormat.INTERLEAVED)
                    plsc.store_scatter(
                        rowsf, [e16, jnp.full((L,), 2 * cc, jnp.int32)],
                        a * w16)
                    plsc.store_scatter(
                        rowsf, [e16, jnp.full((L,), 2 * cc + 1, jnp.int32)],
                        b * w16)
                return 0
            lax.fori_loop(0, 128 // L, scale, 0)
            pltpu.sync_copy(rowsf, acc.at[obat.at[par]], add=True)

        def fire(qbase, limit, nb):
            par = nb % 2

            def bg(k_, _):
                pos = qbase + k_ * L
                lid = pos + lax.iota(jnp.int32, L)
                valid = lid < limit
                spread = (k_ * L) + lax.iota(jnp.int32, L)
                sidx.at[par][pl.ds(k_ * L, L)] = jnp.where(
                    valid, psrc[pl.ds(pos, L)], spread)
                obat.at[par][pl.ds(k_ * L, L)] = jnp.where(
                    valid, poff[pl.ds(pos, L)], spread)
                wbat.at[par][pl.ds(k_ * L, L)] = jnp.where(
                    valid, pex[pl.ds(pos, L)], 0.0)
                return 0
            lax.fori_loop(0, 128 // L, bg, 0)
            pltpu.async_copy(z_hbm.at[sidx.at[par]], rows.at[par], sem)

            @pl.when(nb >= 1)
            def _():
                proc((nb - 1) % 2)
            return nb + 1

        def rngloop(rng, _):
            g = c * (NR // NC) + rng
            base = g * R

            def zi(i, _):
                pltpu.sync_copy(zb, acc.at[pl.ds(s * (R // NS) + i * 16, 16)])
                return 0
            lax.fori_loop(0, R // NS // 16, zi, 0)
            plsc.subcore_barrier()

            def chunk(i, st):
                pcount, nb0 = st
                c0 = (s * ncht + i) * CH2
                pltpu.sync_copy(src_hbm.at[pl.ds(c0, CH2)], srcl)
                pltpu.sync_copy(dst_hbm.at[pl.ds(c0, CH2)], dstl)
                pltpu.sync_copy(ex_hbm.at[pl.ds(c0, CH2)], exl)

                def cmp(j, p):
                    d16 = dstl[pl.ds(j * L, L)]
                    inr = (d16 >= base) & (d16 < base + R)
                    plsc.store_compressed(psrc.at[pl.ds(p, L)],
                                          srcl[pl.ds(j * L, L)], mask=inr)
                    plsc.store_compressed(poff.at[pl.ds(p, L)],
                                          d16 - base, mask=inr)
                    plsc.store_compressed(pex.at[pl.ds(p, L)],
                                          exl[pl.ds(j * L, L)], mask=inr)
                    return p + jnp.sum(inr.astype(jnp.int32))
                p1 = lax.fori_loop(0, CH2 // L, cmp, pcount)

                def drain_cond(qn):
                    return p1 - qn[0] >= 128

                def drain(qn):
                    q, nb = qn
                    nb2 = fire(q, p1, nb)
                    return (q + 128, nb2)
                q1, nb1 = lax.while_loop(drain_cond, drain,
                                         (jnp.int32(0), nb0))

                def mv(k_, _):
                    psrc[pl.ds(k_ * L, L)] = psrc[pl.ds(q1 + k_ * L, L)]
                    poff[pl.ds(k_ * L, L)] = poff[pl.ds(q1 + k_ * L, L)]
                    pex[pl.ds(k_ * L, L)] = pex[pl.ds(q1 + k_ * L, L)]
                    return 0
                lax.fori_loop(0, 128 // L, mv, 0)
                return (p1 - q1, nb1)
            pfin, nbf = lax.fori_loop(0, ncht, chunk,
                                      (jnp.int32(0), jnp.int32(0)))
            nbf2 = fire(jnp.int32(0), pfin, nbf)
            proc((nbf2 - 1) % 2)

            plsc.subcore_barrier()
            pltpu.sync_copy(acc.at[pl.ds(s * (R // NS), R // NS)],
                            num_hbm.at[pl.ds(base + s * (R // NS), R // NS)])
            plsc.subcore_barrier()
            return 0
        lax.fori_loop(0, NR // NC, rngloop, 0)

    return k(src, dst, ex, z, zrows)


def kernel(x_lc, edge_index, batch_lc, enc_W1, enc_b1, enc_W2, enc_b2, enc_W3,
           enc_b3, conv1_W, conv1_as, conv1_ad, conv1_b, norm1_g, norm1_beta,
           conv2_W, conv2_as, conv2_ad, conv2_b, norm2_g, norm2_beta, conv3_W,
           conv3_as, conv3_ad, conv3_b, norm3_g, norm3_beta, out_W1, out_b1,
           out_W2, out_b2, out_W3, out_b3):
    n = x_lc.shape[0]
    e = edge_index.shape[1]
    kw = -(-e // (NW * CH))
    ep = NW * kw * CH

    src = edge_index[0]
    dst = edge_index[1]
    pid = jnp.arange(ep - e, dtype=jnp.int32)
    src_p = jnp.concatenate([src, pid % 1024])
    dst_p = jnp.concatenate([dst, n + (pid % 64)])

    xp = jnp.pad(x_lc, ((0, NP - n), (0, 1)))
    w1 = jnp.pad(enc_W1, ((0, 1), (0, 0)))
    row = lambda v: v.reshape(1, -1)
    col = lambda v: v.reshape(-1, 1)
    zrows = jnp.zeros((16, HID), jnp.float32)

    r, z, asn, adn, sex = _tc_encode(
        xp, w1, row(enc_b1), enc_W2, row(enc_b2), enc_W3, row(enc_b3),
        conv1_W, col(conv1_as), col(conv1_ad))

    layers = [
        (conv1_b, norm1_g, norm1_beta, conv2_W, conv2_as, conv2_ad),
        (conv2_b, norm2_g, norm2_beta, conv3_W, conv3_as, conv3_ad),
    ]
    for cb, g, beta, nw_, nas, nad in layers:
        ex, den = _sc_edge_ex(src_p, dst_p, asn.reshape(NP), adn.reshape(NP), kw)
        z_pk = lax.bitcast_convert_type(
            z.astype(jnp.bfloat16).reshape(NP, HID // 2, 2), jnp.int32)
        num = _sc_aggregate(src_p, dst_p, ex, z_pk, zrows)
        r, z, asn, adn, sex = _tc_mid(
            num, col(den[0]), col(den[1]), sex, z, r,
            row(cb), row(g), row(beta), nw_, col(nas), col(nad))

    ex, den = _sc_edge_ex(src_p, dst_p, asn.reshape(NP), adn.reshape(NP), kw)
    z_pk = lax.bitcast_convert_type(
        z.astype(jnp.bfloat16).reshape(NP, HID // 2, 2), jnp.int32)
    num = _sc_aggregate(src_p, dst_p, ex, z_pk, zrows)
    o = _tc_post(num, col(den[0]), col(den[1]), sex, z, r,
                 row(conv3_b), row(norm3_g), row(norm3_beta),
                 out_W1, row(out_b1), out_W2, row(out_b2), out_W3, row(out_b3))

    return (o[:n], batch_lc)
